# baseline (speedup 1.0000x reference)
import jax
import jax.numpy as jnp
from jax import lax
from jax.experimental import pallas as pl
from jax.experimental.pallas import tpu as pltpu
from jax.experimental.pallas import tpu_sc as plsc

NC = 2
NS = 16
NW = NC * NS

ROWS = 16384
COLS = 32
VOCAB = 1000000
LANES = 16
CHUNK = 128
NCHUNK = ROWS // CHUNK

CH_WORDS = 94720
_full = [CH_WORDS] * 10
_full.append(7812 * 128 - 10 * CH_WORDS)
CH_SIZES = tuple(_full)
TAIL_OFF = 7812 * 128
TAIL = VOCAB - TAIL_OFF


def _body(in_hbm, idx_hbm, tail_hbm, out_hbm, flat_hbm, chunk_v, addr_v, val_v, sem):
    w = lax.axis_index("s") * NC + lax.axis_index("c")

    base = w * VOCAB

    off = 0
    for ln in CH_SIZES:
        pltpu.sync_copy(in_hbm.at[w, pl.ds(off, ln)], chunk_v.at[pl.ds(0, ln)])
        pltpu.sync_copy(chunk_v.at[pl.ds(0, ln)], flat_hbm.at[pl.ds(base + off, ln)])
        off += ln
    pltpu.sync_copy(tail_hbm.at[pl.ds(w * TAIL, TAIL)], chunk_v.at[pl.ds(0, TAIL)])
    pltpu.sync_copy(chunk_v.at[pl.ds(0, TAIL)], flat_hbm.at[pl.ds(base + TAIL_OFF, TAIL)])

    pltpu.sync_copy(idx_hbm.at[w], addr_v)

    def chunk_body(c, carry):
        cbase = c * CHUNK
        for v in range(CHUNK // LANES):
            sl = pl.ds(cbase + v * LANES, LANES)
            addr_v[sl] = addr_v[sl] + base
        return carry

    lax.fori_loop(0, NCHUNK, chunk_body, 0)

    pltpu.async_copy(flat_hbm.at[addr_v], val_v, sem).wait()

    pltpu.sync_copy(val_v, out_hbm.at[w])


@jax.jit
def _gather_rows(in_t, idx_t, tail_flat):
    mesh = plsc.VectorSubcoreMesh(
        core_axis_name="c", subcore_axis_name="s",
        num_cores=NC, num_subcores=NS,
    )
    run = pl.kernel(
        _body,
        mesh=mesh,
        out_type=(
            jax.ShapeDtypeStruct((COLS, ROWS), jnp.float32),
            jax.ShapeDtypeStruct((COLS * VOCAB,), jnp.float32),
        ),
        scratch_types=[
            pltpu.VMEM((CH_WORDS,), jnp.float32),
            pltpu.VMEM((ROWS,), jnp.int32),
            pltpu.VMEM((ROWS,), jnp.float32),
            pltpu.SemaphoreType.DMA,
        ],
    )
    out_t, _ = run(in_t, idx_t, tail_flat)
    return out_t


def kernel(input, index):
    tail_flat = input[TAIL_OFF:, :].T.reshape(-1)
    out_t = _gather_rows(input.T, index.astype(jnp.int32).T, tail_flat)
    return out_t.T


# speedup vs baseline: 1.0503x; 1.0503x over previous
import jax
import jax.numpy as jnp
from jax import lax
from jax.experimental import pallas as pl
from jax.experimental.pallas import tpu as pltpu
from jax.experimental.pallas import tpu_sc as plsc

NC = 2
NS = 16
NW = NC * NS

ROWS = 16384
COLS = 32
VOCAB = 1000000
LANES = 16
CHUNK = 128
NCHUNK = ROWS // CHUNK

CH_WORDS = 46080
CH_SIZES = tuple([CH_WORDS] * 21 + [7812 * 128 - 21 * CH_WORDS])
CH_OFFS = tuple(sum(CH_SIZES[:i]) for i in range(len(CH_SIZES)))
TAIL_OFF = 7812 * 128
TAIL = VOCAB - TAIL_OFF


def _body(in_hbm, idx_hbm, tail_hbm, out_hbm, flat_hbm,
          buf_a, buf_b, addr_v, val_v, in_sem, out_sem, gsem):
    w = lax.axis_index("s") * NC + lax.axis_index("c")
    base = w * VOCAB
    bufs = (buf_a, buf_b)
    n = len(CH_SIZES)

    def fire_in(c):
        return pltpu.async_copy(
            in_hbm.at[w, pl.ds(CH_OFFS[c], CH_SIZES[c])],
            bufs[c % 2].at[pl.ds(0, CH_SIZES[c])],
            in_sem,
        )

    def fire_out(c):
        return pltpu.async_copy(
            bufs[c % 2].at[pl.ds(0, CH_SIZES[c])],
            flat_hbm.at[pl.ds(base + CH_OFFS[c], CH_SIZES[c])],
            out_sem,
        )

    in0 = fire_in(0)

    pltpu.sync_copy(idx_hbm.at[w], addr_v)

    def chunk_body(c, carry):
        cbase = c * CHUNK
        for v in range(CHUNK // LANES):
            sl = pl.ds(cbase + v * LANES, LANES)
            addr_v[sl] = addr_v[sl] + base
        return carry

    lax.fori_loop(0, NCHUNK, chunk_body, 0)

    outs = [None] * n
    ins = [in0] + [None] * (n - 1)
    for c in range(n):
        if c >= 2:
            outs[c - 2].wait()
        if c > 0:
            ins[c] = fire_in(c)
        ins[c].wait()
        outs[c] = fire_out(c)
    if n >= 2:
        outs[n - 2].wait()
    outs[n - 1].wait()

    pltpu.sync_copy(tail_hbm.at[pl.ds(w * TAIL, TAIL)], buf_a.at[pl.ds(0, TAIL)])
    pltpu.sync_copy(buf_a.at[pl.ds(0, TAIL)],
                    flat_hbm.at[pl.ds(base + TAIL_OFF, TAIL)])

    pltpu.async_copy(flat_hbm.at[addr_v], val_v, gsem).wait()

    pltpu.sync_copy(val_v, out_hbm.at[w])


@jax.jit
def _gather_rows(in_t, idx_t, tail_flat):
    mesh = plsc.VectorSubcoreMesh(
        core_axis_name="c", subcore_axis_name="s",
        num_cores=NC, num_subcores=NS,
    )
    run = pl.kernel(
        _body,
        mesh=mesh,
        out_type=(
            jax.ShapeDtypeStruct((COLS, ROWS), jnp.float32),
            jax.ShapeDtypeStruct((COLS * VOCAB,), jnp.float32),
        ),
        scratch_types=[
            pltpu.VMEM((CH_WORDS,), jnp.float32),
            pltpu.VMEM((CH_WORDS,), jnp.float32),
            pltpu.VMEM((ROWS,), jnp.int32),
            pltpu.VMEM((ROWS,), jnp.float32),
            pltpu.SemaphoreType.DMA,
            pltpu.SemaphoreType.DMA,
            pltpu.SemaphoreType.DMA,
        ],
    )
    out_t, _ = run(in_t, idx_t, tail_flat)
    return out_t


def kernel(input, index):
    tail_flat = input[TAIL_OFF:, :].T.reshape(-1)
    out_t = _gather_rows(input.T, index.astype(jnp.int32).T, tail_flat)
    return out_t.T
